# baseline (speedup 1.0000x reference)
import functools

import jax
import jax.numpy as jnp
from jax import lax
from jax.experimental import pallas as pl
from jax.experimental.pallas import tpu as pltpu
from jax.experimental.pallas import tpu_sc as plsc

_TARGET = 0

N_ROWS = 1_000_000
ROW = 8
L = 16
NC, NS = 2, 16
NW = NC * NS
LANE = 128

_CLS, _SCORE = 0, ROW - 1

SC_TILES_PER_W = 144
SC_TILES = SC_TILES_PER_W * NW
CHUNK_TILES = 48
NCHUNKS = SC_TILES_PER_W // CHUNK_TILES
CHUNK_LANES = CHUNK_TILES * LANE
GROUPS_PER_CHUNK = CHUNK_LANES // L
WORDS_PER_W = SC_TILES_PER_W * LANE

TC_START_LANE = SC_TILES * LANE
TC_LANES = N_ROWS - TC_START_LANE
TC_BLOCK = 2048
TC_START_BLK = TC_START_LANE // TC_BLOCK
TC_STEPS = -(-TC_LANES // TC_BLOCK)


def _sc_partials_body(xt_hbm, out_hbm, buf0, buf1, acc_ref, sem0, sem1):
    cid = lax.axis_index("c")
    sid = lax.axis_index("s")
    wid = sid * NC + cid

    base = pl.multiple_of(wid * WORDS_PER_W, LANE)

    bufs = (buf0, buf1)
    sems = (sem0, sem1)

    def start(c):
        src = xt_hbm.at[:, pl.ds(base + c * CHUNK_LANES, CHUNK_LANES)]
        return pltpu.async_copy(src, bufs[c % 2], sems[c % 2])

    def make_group_body(buf):
        def group_body(g, acc):
            cls = buf[_CLS, pl.ds(g * L, L)]
            sc = buf[_SCORE, pl.ds(g * L, L)]
            keep = cls.astype(jnp.int32) == _TARGET
            return acc + jnp.where(keep, sc, 0.0)

        return group_body

    acc = jnp.zeros((L,), jnp.float32)
    cps = [None, None]
    cps[0] = start(0)
    for c in range(NCHUNKS):
        if c + 1 < NCHUNKS:
            cps[(c + 1) % 2] = start(c + 1)
        cps[c % 2].wait()
        acc = plsc.parallel_loop(0, GROUPS_PER_CHUNK, unroll=8, carry=acc)(
            make_group_body(bufs[c % 2])
        )
    acc_ref[...] = acc
    pltpu.sync_copy(acc_ref, out_hbm.at[wid])


_sc_partials = pl.kernel(
    _sc_partials_body,
    out_type=jax.ShapeDtypeStruct((NW, L), jnp.float32),
    mesh=plsc.VectorSubcoreMesh(
        core_axis_name="c", subcore_axis_name="s", num_cores=NC, num_subcores=NS
    ),
    compiler_params=pltpu.CompilerParams(
        needs_layout_passes=False, use_tc_tiling_on_sc=True
    ),
    scratch_types=[
        pltpu.VMEM((ROW, CHUNK_LANES), jnp.float32),
        pltpu.VMEM((ROW, CHUNK_LANES), jnp.float32),
        pltpu.VMEM((L,), jnp.float32),
        pltpu.SemaphoreType.DMA,
        pltpu.SemaphoreType.DMA,
    ],
)


def _tc_reduce_body(x_ref, o_ref, acc_ref):
    i = pl.program_id(0)

    @pl.when(i == 0)
    def _():
        acc_ref[...] = jnp.zeros_like(acc_ref)

    cls = x_ref[_CLS : _CLS + 1, :]
    sc = x_ref[_SCORE : _SCORE + 1, :]
    valid = TC_LANES - i * TC_BLOCK
    lanes = lax.broadcasted_iota(jnp.int32, (1, TC_BLOCK), 1)
    keep = jnp.logical_and(cls.astype(jnp.int32) == _TARGET, lanes < valid)
    acc_ref[...] += jnp.where(keep, sc, 0.0)

    @pl.when(i == TC_STEPS - 1)
    def _():
        o_ref[0, 0] = jnp.sum(acc_ref[...])


_tc_reduce = pl.pallas_call(
    _tc_reduce_body,
    out_shape=jax.ShapeDtypeStruct((1, 1), jnp.float32),
    grid=(TC_STEPS,),
    in_specs=[pl.BlockSpec((ROW, TC_BLOCK), lambda i: (0, TC_START_BLK + i))],
    out_specs=pl.BlockSpec((1, 1), lambda i: (0, 0), memory_space=pltpu.SMEM),
    scratch_shapes=[pltpu.VMEM((1, TC_BLOCK), jnp.float32)],
)


def _finish_body(p_ref, t_ref, o_ref):
    o_ref[0, 0] = -(jnp.sum(p_ref[...]) + t_ref[0, 0])


_finish = pl.pallas_call(
    _finish_body,
    out_shape=jax.ShapeDtypeStruct((1, 1), jnp.float32),
    in_specs=[
        pl.BlockSpec(memory_space=pltpu.VMEM),
        pl.BlockSpec(memory_space=pltpu.SMEM),
    ],
    out_specs=pl.BlockSpec(memory_space=pltpu.SMEM),
)


@jax.jit
def kernel(box3d_branch):
    xt = box3d_branch.T
    partials = _sc_partials(xt)
    tc_part = _tc_reduce(xt)
    return _finish(partials, tc_part)[0, 0]


# speedup vs baseline: 3.8591x; 3.8591x over previous
import functools

import jax
import jax.numpy as jnp
from jax import lax
from jax.experimental import pallas as pl
from jax.experimental.pallas import tpu as pltpu
from jax.experimental.pallas import tpu_sc as plsc

_TARGET = 0

N_ROWS = 1_000_000
ROW = 8
L = 16
NC, NS = 2, 16
NW = NC * NS
LANE = 128

FULL_TILES = N_ROWS // LANE
REM = N_ROWS - FULL_TILES * LANE
TILES_PER_W = FULL_TILES // NW
EXTRA_TILES = FULL_TILES - TILES_PER_W * NW
CHUNK_TILES = 61
NCHUNKS = TILES_PER_W // CHUNK_TILES
CHUNK_LANES = CHUNK_TILES * LANE
GROUPS_PER_CHUNK = CHUNK_LANES // L
WORDS_PER_W = TILES_PER_W * LANE

_CLS, _SCORE = 0, ROW - 1


def _sc_partials_body(
    xt_hbm, rows_hbm, out_hbm, idx_ref, buf0, buf1, tbuf, acc_ref, sem0, sem1
):
    cid = lax.axis_index("c")
    sid = lax.axis_index("s")
    wid = sid * NC + cid

    base = pl.multiple_of(wid * WORDS_PER_W, LANE)
    pltpu.sync_copy(rows_hbm, idx_ref)

    bufs = (buf0, buf1)
    sems = (sem0, sem1)

    def start(c):
        src = xt_hbm.at[idx_ref, pl.ds(base + c * CHUNK_LANES, CHUNK_LANES)]
        return pltpu.async_copy(src, bufs[c % 2], sems[c % 2])

    def make_group_body(buf, cls_row, score_row):
        def group_body(g, acc):
            cls = buf[cls_row, pl.ds(g * L, L)]
            sc = buf[score_row, pl.ds(g * L, L)]
            keep = cls.astype(jnp.int32) == _TARGET
            return acc + jnp.where(keep, sc, 0.0)

        return group_body

    acc = jnp.zeros((L,), jnp.float32)
    cps = [None, None]
    cps[0] = start(0)
    for c in range(NCHUNKS):
        if c + 1 < NCHUNKS:
            cps[(c + 1) % 2] = start(c + 1)
        cps[c % 2].wait()
        acc = plsc.parallel_loop(0, GROUPS_PER_CHUNK, unroll=8, carry=acc)(
            make_group_body(bufs[c % 2], 0, 1)
        )
    acc_ref[...] = acc

    @pl.when(wid < EXTRA_TILES)
    def _():
        off = pl.multiple_of((NW * TILES_PER_W + wid) * LANE, LANE)
        pltpu.sync_copy(xt_hbm.at[:, pl.ds(off, LANE)], tbuf)
        acc_ref[...] = lax.fori_loop(
            0, LANE // L, make_group_body(tbuf, _CLS, _SCORE), acc_ref[...]
        )

    pltpu.sync_copy(acc_ref, out_hbm.at[wid])


_sc_partials = pl.kernel(
    _sc_partials_body,
    out_type=jax.ShapeDtypeStruct((NW, L), jnp.float32),
    mesh=plsc.VectorSubcoreMesh(
        core_axis_name="c", subcore_axis_name="s", num_cores=NC, num_subcores=NS
    ),
    compiler_params=pltpu.CompilerParams(
        needs_layout_passes=False, use_tc_tiling_on_sc=True
    ),
    scratch_types=[
        pltpu.VMEM((2,), jnp.int32),
        pltpu.VMEM((2, CHUNK_LANES), jnp.float32),
        pltpu.VMEM((2, CHUNK_LANES), jnp.float32),
        pltpu.VMEM((ROW, LANE), jnp.float32),
        pltpu.VMEM((L,), jnp.float32),
        pltpu.SemaphoreType.DMA,
        pltpu.SemaphoreType.DMA,
    ],
)


def _finish_body(p_ref, x_ref, o_ref):
    cls = x_ref[_CLS : _CLS + 1, :]
    sc = x_ref[_SCORE : _SCORE + 1, :]
    valid = lax.broadcasted_iota(jnp.int32, (1, LANE), 1) < REM
    keep = jnp.logical_and(cls.astype(jnp.int32) == _TARGET, valid)
    tail = jnp.sum(jnp.where(keep, sc, 0.0))
    o_ref[0, 0] = -(jnp.sum(p_ref[...]) + tail)


_finish = pl.pallas_call(
    _finish_body,
    out_shape=jax.ShapeDtypeStruct((1, 1), jnp.float32),
    grid=(1,),
    in_specs=[
        pl.BlockSpec((NW, L), lambda i: (0, 0)),
        pl.BlockSpec((ROW, LANE), lambda i: (0, FULL_TILES)),
    ],
    out_specs=pl.BlockSpec((1, 1), lambda i: (0, 0), memory_space=pltpu.SMEM),
)


@jax.jit
def kernel(box3d_branch):
    xt = box3d_branch.T
    rows = jnp.array([_CLS, _SCORE], dtype=jnp.int32)
    partials = _sc_partials(xt, rows)
    return _finish(partials, xt)[0, 0]
